# baseline (speedup 1.0000x reference)
import jax
import jax.numpy as jnp
from jax import lax
from jax.experimental import pallas as pl
from jax.experimental.pallas import tpu as pltpu
from jax.experimental.pallas import tpu_sc as plsc

B = 16384
D = 64
ROW2 = 2 * D
NUM_CORES = 2
NUM_SUBCORES = 16
NUM_WORKERS = NUM_CORES * NUM_SUBCORES
BPW = B // NUM_WORKERS
CHUNK = 128
NCHUNK = BPW // CHUNK
LANES = 16
GROUPS = CHUNK // LANES


def _bpr_body(users_hbm, item_i_hbm, item_j_hbm, user2_hbm, item2_hbm,
              pos_hbm, neg_hbm,
              idx_u, idx_i, idx_j, row_u, row_i, row_j,
              stage, pos_v, neg_v, sem0, sem1):
    cid = lax.axis_index("c")
    sid = lax.axis_index("s")
    wid = sid * NUM_CORES + cid
    base = wid * BPW

    pltpu.sync_copy(users_hbm.at[pl.ds(base, BPW)], idx_u)
    pltpu.sync_copy(item_i_hbm.at[pl.ds(base, BPW)], idx_i)
    pltpu.sync_copy(item_j_hbm.at[pl.ds(base, BPW)], idx_j)
    for g in range(BPW // LANES):
        sl = pl.ds(g * LANES, LANES)
        row_u[sl] = idx_u[sl] >> 1
        row_i[sl] = idx_i[sl] >> 1
        row_j[sl] = idx_j[sl] >> 1

    sems = (sem0, sem1)

    def fire(c):
        buf = c % 2
        sl = pl.ds(c * CHUNK, CHUNK)
        sem = sems[buf]
        pltpu.async_copy(user2_hbm.at[row_u.at[sl]], stage.at[buf, 0], sem)
        pltpu.async_copy(item2_hbm.at[row_i.at[sl]], stage.at[buf, 1], sem)
        pltpu.async_copy(item2_hbm.at[row_j.at[sl]], stage.at[buf, 2], sem)

    def drain(c):
        buf = c % 2
        sem = sems[buf]
        for t in range(3):
            pltpu.make_async_copy(
                user2_hbm.at[pl.ds(0, CHUNK), :], stage.at[buf, t], sem).wait()

    lane = lax.iota(jnp.int32, LANES)
    fire(0)
    for c in range(NCHUNK):
        if c + 1 < NCHUNK:
            fire(c + 1)
        drain(c)
        buf = c % 2
        su = stage.at[buf, 0]
        si = stage.at[buf, 1]
        sj = stage.at[buf, 2]

        def group_body(g, _):
            b0 = c * CHUNK + g * LANES
            sl = pl.ds(b0, LANES)
            rows = g * LANES + lane
            off_u = (idx_u[sl] & 1) * D
            off_i = (idx_i[sl] & 1) * D
            off_j = (idx_j[sl] & 1) * D
            accp = jnp.zeros((LANES,), jnp.float32)
            accn = jnp.zeros((LANES,), jnp.float32)
            for d in range(D):
                u = plsc.load_gather(su, [rows, off_u + d])
                iv = plsc.load_gather(si, [rows, off_i + d])
                jv = plsc.load_gather(sj, [rows, off_j + d])
                accp = accp + u * iv
                accn = accn + u * jv
            pos_v[sl] = accp
            neg_v[sl] = accn
            return ()

        lax.fori_loop(0, GROUPS, group_body, ())

    out = pl.ds(base, BPW)
    pltpu.sync_copy(pos_v, pos_hbm.at[out])
    pltpu.sync_copy(neg_v, neg_hbm.at[out])


@jax.jit
def _bpr(users, item_i, item_j, user2, item2):
    mesh = plsc.VectorSubcoreMesh(core_axis_name="c", subcore_axis_name="s")
    f = pl.kernel(
        _bpr_body,
        out_type=(
            jax.ShapeDtypeStruct((B,), jnp.float32),
            jax.ShapeDtypeStruct((B,), jnp.float32),
        ),
        mesh=mesh,
        compiler_params=pltpu.CompilerParams(
            needs_layout_passes=False, use_tc_tiling_on_sc=True),
        scratch_types=[
            pltpu.VMEM((BPW,), jnp.int32),
            pltpu.VMEM((BPW,), jnp.int32),
            pltpu.VMEM((BPW,), jnp.int32),
            pltpu.VMEM((BPW,), jnp.int32),
            pltpu.VMEM((BPW,), jnp.int32),
            pltpu.VMEM((BPW,), jnp.int32),
            pltpu.VMEM((2, 3, CHUNK, ROW2), jnp.float32),
            pltpu.VMEM((BPW,), jnp.float32),
            pltpu.VMEM((BPW,), jnp.float32),
            pltpu.SemaphoreType.DMA,
            pltpu.SemaphoreType.DMA,
        ],
    )
    return f(users, item_i, item_j, user2, item2)


def kernel(users, item_i, item_j, user_emb, item_emb):
    user2 = user_emb.reshape(user_emb.shape[0] // 2, ROW2)
    item2 = item_emb.reshape(item_emb.shape[0] // 2, ROW2)
    return _bpr(users.astype(jnp.int32), item_i.astype(jnp.int32),
                item_j.astype(jnp.int32), user2, item2)


# speedup vs baseline: 1.0522x; 1.0522x over previous
import jax
import jax.numpy as jnp
from jax import lax
from jax.experimental import pallas as pl
from jax.experimental.pallas import tpu as pltpu
from jax.experimental.pallas import tpu_sc as plsc

B = 16384
D = 64
ROW2 = 2 * D
NUM_CORES = 2
NUM_SUBCORES = 16
NUM_WORKERS = NUM_CORES * NUM_SUBCORES
BPW = B // NUM_WORKERS
CHUNK = 128
NCHUNK = BPW // CHUNK
LANES = 16
GROUPS = CHUNK // LANES


def _bpr_body(users_hbm, item_i_hbm, item_j_hbm, user2_hbm, item2_hbm,
              pos_hbm, neg_hbm,
              idx_u, idx_i, idx_j, stage, pos_v, neg_v, sem0, sem1):
    cid = lax.axis_index("c")
    sid = lax.axis_index("s")
    wid = sid * NUM_CORES + cid
    base = wid * BPW

    pltpu.sync_copy(users_hbm.at[pl.ds(base, BPW)], idx_u)
    pltpu.sync_copy(item_i_hbm.at[pl.ds(base, BPW)], idx_i)
    pltpu.sync_copy(item_j_hbm.at[pl.ds(base, BPW)], idx_j)

    sems = (sem0, sem1)

    def fire(c):
        buf = c % 2
        sl = pl.ds(c * CHUNK, CHUNK)
        sem = sems[buf]
        pltpu.async_copy(user2_hbm.at[idx_u.at[sl]], stage.at[buf, 0], sem)
        pltpu.async_copy(item2_hbm.at[idx_i.at[sl]], stage.at[buf, 1], sem)
        pltpu.async_copy(item2_hbm.at[idx_j.at[sl]], stage.at[buf, 2], sem)

    def drain(c):
        buf = c % 2
        sem = sems[buf]
        for t in range(3):
            pltpu.make_async_copy(
                user2_hbm.at[pl.ds(0, CHUNK), :], stage.at[buf, t], sem).wait()

    lane = lax.iota(jnp.int32, LANES)
    fire(0)
    for c in range(NCHUNK):
        if c + 1 < NCHUNK:
            fire(c + 1)
        drain(c)
        buf = c % 2
        su = stage.at[buf, 0]
        si = stage.at[buf, 1]
        sj = stage.at[buf, 2]

        def group_body(g, _):
            b0 = c * CHUNK + g * LANES
            sl = pl.ds(b0, LANES)
            rows = g * LANES + lane
            accp = jnp.zeros((LANES,), jnp.float32)
            accn = jnp.zeros((LANES,), jnp.float32)
            for d in range(D):
                col = jnp.full((LANES,), d, jnp.int32)
                u = plsc.load_gather(su, [rows, col])
                iv = plsc.load_gather(si, [rows, col])
                jv = plsc.load_gather(sj, [rows, col])
                accp = accp + u * iv
                accn = accn + u * jv
            pos_v[sl] = accp
            neg_v[sl] = accn
            return ()

        lax.fori_loop(0, GROUPS, group_body, ())

    out = pl.ds(base, BPW)
    pltpu.sync_copy(pos_v, pos_hbm.at[out])
    pltpu.sync_copy(neg_v, neg_hbm.at[out])


@jax.jit
def _bpr(users, item_i, item_j, user2, item2):
    mesh = plsc.VectorSubcoreMesh(core_axis_name="c", subcore_axis_name="s")
    f = pl.kernel(
        _bpr_body,
        out_type=(
            jax.ShapeDtypeStruct((B,), jnp.float32),
            jax.ShapeDtypeStruct((B,), jnp.float32),
        ),
        mesh=mesh,
        compiler_params=pltpu.CompilerParams(
            needs_layout_passes=False, use_tc_tiling_on_sc=True),
        scratch_types=[
            pltpu.VMEM((BPW,), jnp.int32),
            pltpu.VMEM((BPW,), jnp.int32),
            pltpu.VMEM((BPW,), jnp.int32),
            pltpu.VMEM((2, 3, CHUNK, ROW2), jnp.float32),
            pltpu.VMEM((BPW,), jnp.float32),
            pltpu.VMEM((BPW,), jnp.float32),
            pltpu.SemaphoreType.DMA,
            pltpu.SemaphoreType.DMA,
        ],
    )
    return f(users, item_i, item_j, user2, item2)


def kernel(users, item_i, item_j, user_emb, item_emb):
    user2 = jnp.pad(user_emb, ((0, 0), (0, ROW2 - D)))
    item2 = jnp.pad(item_emb, ((0, 0), (0, ROW2 - D)))
    return _bpr(users.astype(jnp.int32), item_i.astype(jnp.int32),
                item_j.astype(jnp.int32), user2, item2)


# speedup vs baseline: 2.1792x; 2.0711x over previous
import jax
import jax.numpy as jnp
from jax import lax
from jax.experimental import pallas as pl
from jax.experimental.pallas import tpu as pltpu
from jax.experimental.pallas import tpu_sc as plsc

B = 16384
D = 64
BLK = 8
NUM_CORES = 2
NUM_SUBCORES = 16
NUM_WORKERS = NUM_CORES * NUM_SUBCORES
BPW = B // NUM_WORKERS
CHUNK = 16
NCHUNK = BPW // CHUNK
LANES = 16


def _bpr_body(users_hbm, item_i_hbm, item_j_hbm, user3_hbm, item3_hbm,
              pos_hbm, neg_hbm,
              idx_u, idx_i, idx_j, stage, pos_v, neg_v, sem0, sem1):
    cid = lax.axis_index("c")
    sid = lax.axis_index("s")
    wid = sid * NUM_CORES + cid
    base = wid * BPW

    pltpu.sync_copy(users_hbm.at[pl.ds(base, BPW)], idx_u)
    pltpu.sync_copy(item_i_hbm.at[pl.ds(base, BPW)], idx_i)
    pltpu.sync_copy(item_j_hbm.at[pl.ds(base, BPW)], idx_j)

    sems = (sem0, sem1)
    lane = lax.iota(jnp.int32, LANES)

    def fire(c, buf):
        sl = pl.ds(c * CHUNK, LANES)
        sem = sems[buf]
        for t, (idx, tab) in enumerate(
                ((idx_u, user3_hbm), (idx_i, item3_hbm), (idx_j, item3_hbm))):
            blocks = idx[sl] >> 3
            for e in range(LANES):
                blk = jnp.sum(jnp.where(lane == e, blocks, 0))
                pltpu.async_copy(tab.at[blk], stage.at[buf, t, e], sem)

    def drain(buf):
        sem = sems[buf]
        for t in range(3):
            for e in range(LANES):
                pltpu.make_async_copy(
                    user3_hbm.at[0], stage.at[buf, t, e], sem).wait()

    def compute(c, buf):
        sl = pl.ds(c * CHUNK, LANES)
        sub_u = idx_u[sl] & 7
        sub_i = idx_i[sl] & 7
        sub_j = idx_j[sl] & 7
        su = stage.at[buf, 0]
        si = stage.at[buf, 1]
        sj = stage.at[buf, 2]
        accp = jnp.zeros((LANES,), jnp.float32)
        accn = jnp.zeros((LANES,), jnp.float32)
        for d in range(D):
            col = jnp.full((LANES,), d, jnp.int32)
            u = plsc.load_gather(su, [lane, sub_u, col])
            iv = plsc.load_gather(si, [lane, sub_i, col])
            jv = plsc.load_gather(sj, [lane, sub_j, col])
            accp = accp + u * iv
            accn = accn + u * jv
        pos_v[sl] = accp
        neg_v[sl] = accn

    fire(0, 0)

    def pair_body(k, _):
        c0 = 2 * k
        fire(c0 + 1, 1)
        drain(0)
        compute(c0, 0)
        fire(jnp.minimum(c0 + 2, NCHUNK - 1), 0)
        drain(1)
        compute(c0 + 1, 1)
        return ()

    lax.fori_loop(0, NCHUNK // 2 - 1, pair_body, ())
    c0 = NCHUNK - 2
    fire(c0 + 1, 1)
    drain(0)
    compute(c0, 0)
    drain(1)
    compute(c0 + 1, 1)

    out = pl.ds(base, BPW)
    pltpu.sync_copy(pos_v, pos_hbm.at[out])
    pltpu.sync_copy(neg_v, neg_hbm.at[out])


@jax.jit
def _bpr(users, item_i, item_j, user3, item3):
    mesh = plsc.VectorSubcoreMesh(core_axis_name="c", subcore_axis_name="s")
    f = pl.kernel(
        _bpr_body,
        out_type=(
            jax.ShapeDtypeStruct((B,), jnp.float32),
            jax.ShapeDtypeStruct((B,), jnp.float32),
        ),
        mesh=mesh,
        compiler_params=pltpu.CompilerParams(
            needs_layout_passes=False, use_tc_tiling_on_sc=True),
        scratch_types=[
            pltpu.VMEM((BPW,), jnp.int32),
            pltpu.VMEM((BPW,), jnp.int32),
            pltpu.VMEM((BPW,), jnp.int32),
            pltpu.VMEM((2, 3, CHUNK, BLK, D), jnp.float32),
            pltpu.VMEM((BPW,), jnp.float32),
            pltpu.VMEM((BPW,), jnp.float32),
            pltpu.SemaphoreType.DMA,
            pltpu.SemaphoreType.DMA,
        ],
    )
    return f(users, item_i, item_j, user3, item3)


def kernel(users, item_i, item_j, user_emb, item_emb):
    user3 = user_emb.reshape(user_emb.shape[0] // BLK, BLK, D)
    item3 = item_emb.reshape(item_emb.shape[0] // BLK, BLK, D)
    return _bpr(users.astype(jnp.int32), item_i.astype(jnp.int32),
                item_j.astype(jnp.int32), user3, item3)
